# Initial kernel scaffold; baseline (speedup 1.0000x reference)
import jax
import jax.numpy as jnp
from jax.experimental import pallas as pl


def kernel(data):
    raise NotImplementedError("write your pallas kernel here")


# speedup vs baseline: 1050.9395x; 1050.9395x over previous
import functools

import numpy as np
import jax
import jax.numpy as jnp
from jax import lax
from jax.experimental import pallas as pl
from jax.experimental.pallas import tpu as pltpu
from jax.experimental.pallas import tpu_sc as plsc

_H = 2048
_W = 2048
_CH = 4
_NWORK = 32
_RPW = _H // _NWORK
_NCHUNK = _W // 16

_NORM = np.sqrt(np.float32(4.0 * 4.0 + 1024.0 * 1024.0 + 1024.0 * 1024.0),
                dtype=np.float32)
_S = np.float32(np.float32(_NORM) * np.float32(0.5) / np.float32(2048.0))


def _trig_kernel(cos_ref, sin_ref):
    i = lax.broadcasted_iota(jnp.int32, (16, 128), 0)
    j = lax.broadcasted_iota(jnp.int32, (16, 128), 1)
    t = (i * 128 + j).astype(jnp.float32)
    ang = t * 2.0 * np.float32(np.pi) / 2048.0
    cos_ref[...] = jnp.cos(ang)
    sin_ref[...] = jnp.sin(ang)


_trig = pl.pallas_call(
    _trig_kernel,
    out_shape=(jax.ShapeDtypeStruct((16, 128), jnp.float32),
               jax.ShapeDtypeStruct((16, 128), jnp.float32)),
)

_mesh = plsc.VectorSubcoreMesh(core_axis_name="c", subcore_axis_name="s")


@functools.partial(
    pl.kernel,
    mesh=_mesh,
    out_type=jax.ShapeDtypeStruct((_CH, _H, _W), jnp.float32),
    scratch_types=[
        pltpu.VMEM((_CH * 4 * 1024,), jnp.float32),
        pltpu.VMEM((_RPW, 16), jnp.float32),
        pltpu.VMEM((_RPW, 16), jnp.float32),
        pltpu.VMEM((2, _CH, 1, _W), jnp.float32),
        pltpu.SemaphoreType.DMA,
        pltpu.SemaphoreType.DMA,
    ],
    compiler_params=pltpu.CompilerParams(needs_layout_passes=False),
)
def _remap(tbl_hbm, cosb_hbm, sinb_hbm, out_hbm,
           table_v, cos_v, sin_v, buf_v, sem0, sem1):
    wid = lax.axis_index("s") * 2 + lax.axis_index("c")
    base = wid * _RPW
    pltpu.sync_copy(tbl_hbm, table_v)
    pltpu.sync_copy(cosb_hbm.at[pl.ds(base, _RPW)], cos_v)
    pltpu.sync_copy(sinb_hbm.at[pl.ds(base, _RPW)], sin_v)
    iota16 = lax.iota(jnp.int32, 16)
    sems = (sem0, sem1)

    def group(g, carry):
        for b in range(2):
            rl = 2 * g + b
            row = base + rl

            @pl.when(g > 0)
            def _wait():
                pltpu.make_async_copy(out_hbm.at[:, pl.ds(row, 1), :],
                                      buf_v.at[b], sems[b]).wait()

            cv = cos_v[rl]
            sv = sin_v[rl]

            def chunk(k, c2):
                rrv = (iota16 + k * 16).astype(jnp.float32)
                rho = rrv * _S
                x = 512.0 + rho * cv
                y = 2.0 - rho * sv
                m = (x >= 0.0) & (x < 1024.0) & (y >= 0.0) & (y < 4.0)
                xi = jnp.clip(x.astype(jnp.int32), 0, 1023)
                yi = jnp.clip(y.astype(jnp.int32), 0, 3)
                idx = yi * 1024 + xi
                for c in range(_CH):
                    val = plsc.load_gather(table_v, [idx + (c * 4096)])
                    buf_v[b, c, 0, pl.ds(k * 16, 16)] = jnp.where(m, val, 0.0)
                return c2

            lax.fori_loop(0, _NCHUNK, chunk, 0)
            pltpu.async_copy(buf_v.at[b], out_hbm.at[:, pl.ds(row, 1), :],
                             sems[b])
        return carry

    lax.fori_loop(0, _RPW // 2, group, 0)
    for b in range(2):
        pltpu.make_async_copy(out_hbm.at[:, pl.ds(base, 1), :],
                              buf_v.at[b], sems[b]).wait()


def kernel(data):
    cos_t, sin_t = _trig()
    cos_b = jnp.broadcast_to(cos_t.reshape(_H, 1), (_H, 16))
    sin_b = jnp.broadcast_to(sin_t.reshape(_H, 1), (_H, 16))
    tbl = data[:, :4, :].reshape(_CH * 4 * 1024)
    return _remap(tbl, cos_b, sin_b)


# speedup vs baseline: 3461.1303x; 3.2934x over previous
import functools

import numpy as np
import jax
import jax.numpy as jnp
from jax import lax
from jax.experimental import pallas as pl
from jax.experimental.pallas import tpu as pltpu
from jax.experimental.pallas import tpu_sc as plsc

_H = 2048
_W = 2048
_CH = 4
_NWORK = 32
_RPW = _H // _NWORK
_NCHUNK = _W // 16

_NORM = np.sqrt(np.float32(4.0 * 4.0 + 1024.0 * 1024.0 + 1024.0 * 1024.0),
                dtype=np.float32)
_S = np.float32(np.float32(_NORM) * np.float32(0.5) / np.float32(2048.0))


def _trig_kernel(cos_ref, sin_ref, nv_ref):
    i = lax.broadcasted_iota(jnp.int32, (16, 128), 0)
    j = lax.broadcasted_iota(jnp.int32, (16, 128), 1)
    t = (i * 128 + j).astype(jnp.float32)
    ang = t * 2.0 * np.float32(np.pi) / 2048.0
    c = jnp.cos(ang)
    s = jnp.sin(ang)
    cos_ref[...] = c
    sin_ref[...] = s
    asc = jnp.abs(c) * _S
    ass = jnp.abs(s) * _S
    bx = jnp.where(asc > 0.0, 512.0 / jnp.maximum(asc, 1e-30), 3000.0)
    by = jnp.where(ass > 0.0, 2.0 / jnp.maximum(ass, 1e-30), 3000.0)
    bound = jnp.minimum(jnp.minimum(bx, by), 3000.0)
    nv = jnp.clip((bound * (1.0 / 16.0)).astype(jnp.int32) + 3, 1, _NCHUNK)
    nv_ref[...] = nv


_trig = pl.pallas_call(
    _trig_kernel,
    out_shape=(jax.ShapeDtypeStruct((16, 128), jnp.float32),
               jax.ShapeDtypeStruct((16, 128), jnp.float32),
               jax.ShapeDtypeStruct((16, 128), jnp.int32)),
)

_mesh = plsc.VectorSubcoreMesh(core_axis_name="c", subcore_axis_name="s")


@functools.partial(
    pl.kernel,
    mesh=_mesh,
    out_type=jax.ShapeDtypeStruct((_CH, _H, _W), jnp.float32),
    scratch_types=[
        pltpu.VMEM((_CH * 4 * 1024,), jnp.float32),
        pltpu.VMEM((_RPW, 16), jnp.float32),
        pltpu.VMEM((_RPW, 16), jnp.float32),
        pltpu.VMEM((_RPW, 16), jnp.int32),
        pltpu.VMEM((2, _CH, 1, _W), jnp.float32),
        pltpu.SemaphoreType.DMA,
        pltpu.SemaphoreType.DMA,
    ],
    compiler_params=pltpu.CompilerParams(needs_layout_passes=False),
)
def _remap(tbl_hbm, cosb_hbm, sinb_hbm, nvb_hbm, out_hbm,
           table_v, cos_v, sin_v, nv_v, buf_v, sem0, sem1):
    wid = lax.axis_index("s") * 2 + lax.axis_index("c")
    base = wid * _RPW
    pltpu.sync_copy(tbl_hbm, table_v)
    pltpu.sync_copy(cosb_hbm.at[pl.ds(base, _RPW)], cos_v)
    pltpu.sync_copy(sinb_hbm.at[pl.ds(base, _RPW)], sin_v)
    pltpu.sync_copy(nvb_hbm.at[pl.ds(base, _RPW)], nv_v)
    iota16 = lax.iota(jnp.int32, 16)
    zeros16 = jnp.zeros((16,), jnp.float32)
    sems = (sem0, sem1)

    def group(g, carry):
        carry = list(carry)
        for b in range(2):
            rl = 2 * g + b
            row = base + rl

            @pl.when(g > 0)
            def _wait():
                pltpu.make_async_copy(out_hbm.at[:, pl.ds(row, 1), :],
                                      buf_v.at[b], sems[b]).wait()

            def zchunk(k, c2):
                for c in range(_CH):
                    buf_v[b, c, 0, pl.ds(k * 16, 16)] = zeros16
                return c2

            lax.fori_loop(0, carry[b], zchunk, 0)

            cv = cos_v[rl]
            sv = sin_v[rl]
            nv = jnp.max(nv_v[rl])

            def chunk(k, c2):
                rrv = (iota16 + k * 16).astype(jnp.float32)
                rho = rrv * _S
                x = 512.0 + rho * cv
                y = 2.0 - rho * sv
                m = (x >= 0.0) & (x < 1024.0) & (y >= 0.0) & (y < 4.0)
                xi = jnp.clip(x.astype(jnp.int32), 0, 1023)
                yi = jnp.clip(y.astype(jnp.int32), 0, 3)
                idx = yi * 1024 + xi
                for c in range(_CH):
                    val = plsc.load_gather(table_v, [idx + (c * 4096)])
                    buf_v[b, c, 0, pl.ds(k * 16, 16)] = jnp.where(m, val, 0.0)
                return c2

            lax.fori_loop(0, nv, chunk, 0)
            pltpu.async_copy(buf_v.at[b], out_hbm.at[:, pl.ds(row, 1), :],
                             sems[b])
            carry[b] = nv
        return tuple(carry)

    lax.fori_loop(0, _RPW // 2, group,
                  (jnp.int32(_NCHUNK), jnp.int32(_NCHUNK)))
    for b in range(2):
        pltpu.make_async_copy(out_hbm.at[:, pl.ds(base, 1), :],
                              buf_v.at[b], sems[b]).wait()


def kernel(data):
    cos_t, sin_t, nv_t = _trig()
    cos_b = jnp.broadcast_to(cos_t.reshape(_H, 1), (_H, 16))
    sin_b = jnp.broadcast_to(sin_t.reshape(_H, 1), (_H, 16))
    nv_b = jnp.broadcast_to(nv_t.reshape(_H, 1), (_H, 16))
    tbl = data[:, :4, :].reshape(_CH * 4 * 1024)
    return _remap(tbl, cos_b, sin_b, nv_b)
